# baseline (speedup 1.0000x reference)
import functools

import jax
import jax.numpy as jnp
from jax import lax
from jax.experimental import pallas as pl
from jax.experimental.pallas import tpu as pltpu
from jax.experimental.pallas import tpu_sc as plsc

R = 128
VMIN = -3.0
VMAX = 3.0

NC, NS, L = 2, 16, 16
NW = NC * NS
V = R * R * R
TW = 16
CS = 1024
NB = 4
C = 2048

_SC_PARAMS = pltpu.CompilerParams(needs_layout_passes=False,
                                  use_tc_tiling_on_sc=False)


def _wid():
    return lax.axis_index("s") * NC + lax.axis_index("c")


_SLAB = CS + 136


def _build_body(src_hbm, tab_hbm, src_v, dst_v, sin, sout):
    rpw = V // NW
    base = _wid() * rpw
    nchunk = rpw // CS
    lanes = lax.iota(jnp.int32, L)

    def slab_start(a, dx, c):
        x = a // (R * R)
        yb = (a - x * R * R) // R
        xs = jnp.minimum(x + dx, R - 1)
        return pl.multiple_of(xs * (R * R * 3) + c * (R * R) + yb * R, 128)

    def stage(ci, sl):
        a = base + ci * CS
        for dx in (0, 1):
            for c in range(3):
                pltpu.async_copy(
                    src_hbm.at[pl.ds(slab_start(a, dx, c), _SLAB)],
                    src_v.at[sl, dx * 3 + c], sin.at[sl])

    def wait_stage(ci, sl):
        a = base + ci * CS
        for dx in (0, 1):
            for c in range(3):
                pltpu.make_async_copy(
                    src_hbm.at[pl.ds(slab_start(a, dx, c), _SLAB)],
                    src_v.at[sl, dx * 3 + c], sin.at[sl]).wait()

    def compute(ci, sl):
        a = base + ci * CS

        slv = jnp.full((L,), sl, jnp.int32)

        def yline(yl, _):
            for zg in range(8):
                zb = zg * L
                rows = (yl * 8 + zg) * L + lanes
                for dx in (0, 1):
                    for c in range(3):
                        s = dx * 3 + c
                        colw = jnp.full((L,), dx * 6 + c, jnp.int32)
                        for dy in (0, 1):
                            off = (yl + dy) * R + zb
                            v0 = src_v[sl, s, pl.ds(off, L)]
                            v1 = src_v[sl, s, pl.ds(off + 1, L)]
                            pair = plsc.bitcast(
                                plsc.pack(v0, v1,
                                          format=plsc.PackFormat.INTERLEAVED),
                                jnp.float32)
                            plsc.store_scatter(dst_v,
                                               [slv, rows, colw + 3 * dy],
                                               pair)
            return 0

        lax.fori_loop(0, CS // L // 8, yline, 0)
        pltpu.async_copy(dst_v.at[sl], tab_hbm.at[pl.ds(a, CS)], sout.at[sl])

    def wait_out(ci, sl):
        a = base + ci * CS
        pltpu.make_async_copy(dst_v.at[sl], tab_hbm.at[pl.ds(a, CS)],
                              sout.at[sl]).wait()

    stage(0, 0)

    def pipe_step(ci, _):
        sl = lax.rem(ci, 2)
        nsl = lax.rem(ci + 1, 2)
        osl = lax.rem(ci, NB)

        @pl.when(ci + 1 < nchunk)
        def _():
            stage(ci + 1, nsl)

        wait_stage(ci, sl)

        @pl.when(ci >= NB)
        def _():
            wait_out(ci - NB, osl)

        compute(ci, osl)
        return 0

    lax.fori_loop(0, nchunk, pipe_step, 0)
    for k in range(NB):
        ci = nchunk - NB + k
        wait_out(ci, lax.rem(ci, NB))


def _sample_body(idx_hbm, fx_hbm, fy_hbm, fz_hbm, tab_hbm,
                 r_hbm, g_hbm, b_hbm,
                 idx_v, f_v, rows_v, out_v, sems, osems, *, bpw):
    wbase = _wid() * bpw
    nchunk = bpw // C
    lanes = lax.iota(jnp.int32, L)

    def stage_fire(ci, sl):
        base = wbase + ci * C
        pltpu.sync_copy(idx_hbm.at[pl.ds(base, C)], idx_v.at[sl])
        pltpu.sync_copy(fx_hbm.at[pl.ds(base, C)], f_v.at[sl, 0])
        pltpu.sync_copy(fy_hbm.at[pl.ds(base, C)], f_v.at[sl, 1])
        pltpu.sync_copy(fz_hbm.at[pl.ds(base, C)], f_v.at[sl, 2])
        pltpu.async_copy(tab_hbm.at[idx_v.at[sl]], rows_v.at[sl],
                         sems.at[sl])

    def wait_gathers(sl):
        pltpu.make_async_copy(tab_hbm.at[idx_v.at[sl]],
                              rows_v.at[sl], sems.at[sl]).wait()

    def blend_write(ci, sl):
        base = wbase + ci * C
        slv = jnp.full((L,), sl, jnp.int32)

        def blend_group(g, _):
            o = g * L
            rows = o + lanes
            fx = f_v[sl, 0, pl.ds(o, L)]
            fy = f_v[sl, 1, pl.ds(o, L)]
            fz = f_v[sl, 2, pl.ds(o, L)]
            gz = 1 - fz
            gy = 1 - fy
            gx = 1 - fx
            for c in range(3):

                def zlerp(dx, dy):
                    word = plsc.load_gather(
                        rows_v, [slv, rows,
                                 jnp.full((L,), dx * 6 + dy * 3 + c,
                                          jnp.int32)])
                    z0, z1 = plsc.unpack(
                        plsc.bitcast(word, jnp.bfloat16),
                        format=plsc.PackFormat.INTERLEAVED)
                    return z0 * gz + z1 * fz

                c0 = zlerp(0, 0) * gy + zlerp(0, 1) * fy
                c1 = zlerp(1, 0) * gy + zlerp(1, 1) * fy
                out_v[sl, c, pl.ds(o, L)] = c0 * gx + c1 * fx
            return 0

        lax.fori_loop(0, C // L, blend_group, 0)
        for ch, hbm in enumerate((r_hbm, g_hbm, b_hbm)):
            pltpu.async_copy(out_v.at[sl, ch], hbm.at[pl.ds(base, C)],
                             osems.at[sl])

    def wait_out(ci, sl):
        base = wbase + ci * C
        for ch, hbm in enumerate((r_hbm, g_hbm, b_hbm)):
            pltpu.make_async_copy(out_v.at[sl, ch], hbm.at[pl.ds(base, C)],
                                  osems.at[sl]).wait()

    stage_fire(0, 0)

    def pipe_step(ci, _):
        sl = lax.rem(ci, 2)
        nsl = lax.rem(ci + 1, 2)

        @pl.when(ci + 1 < nchunk)
        def _():
            stage_fire(ci + 1, nsl)

        wait_gathers(sl)

        @pl.when(ci >= 2)
        def _():
            wait_out(ci - 2, sl)

        blend_write(ci, sl)
        return 0

    lax.fori_loop(0, nchunk, pipe_step, 0)
    wait_out(nchunk - 2, lax.rem(nchunk - 2, 2))
    wait_out(nchunk - 1, lax.rem(nchunk - 1, 2))


def kernel(position, grid):
    b = position.shape[0]
    assert b % (NW * C) == 0
    bpw = b // NW
    mesh = plsc.VectorSubcoreMesh(core_axis_name="c", subcore_axis_name="s",
                                  num_cores=NC, num_subcores=NS)

    t = (position - VMIN) / (VMAX - VMIN) * (R - 1)
    t = jnp.clip(t, 0.0, R - 1 - 1e-6)
    i0 = jnp.floor(t).astype(jnp.int32)
    f = t - i0.astype(jnp.float32)
    cbase = (i0[:, 0] * R + i0[:, 1]) * R + i0[:, 2]
    fx, fy, fz = f[:, 0], f[:, 1], f[:, 2]

    gwords = jnp.concatenate(
        [jnp.transpose(grid, (0, 3, 1, 2)).reshape(-1),
         jnp.zeros((512,), jnp.float32)])

    table = pl.kernel(
        _build_body,
        out_type=jax.ShapeDtypeStruct((V, TW), jnp.float32),
        mesh=mesh,
        scratch_types=[
            pltpu.VMEM((2, 6, _SLAB), jnp.float32),
            pltpu.VMEM((NB, CS, TW), jnp.float32),
            pltpu.SemaphoreType.DMA((2,)),
            pltpu.SemaphoreType.DMA((NB,)),
        ],
        compiler_params=_SC_PARAMS,
    )(gwords)

    rgb = pl.kernel(
        functools.partial(_sample_body, bpw=bpw),
        out_type=[jax.ShapeDtypeStruct((b,), jnp.float32)] * 3,
        mesh=mesh,
        scratch_types=[
            pltpu.VMEM((2, C), jnp.int32),
            pltpu.VMEM((2, 3, C), jnp.float32),
            pltpu.VMEM((2, C, TW), jnp.float32),
            pltpu.VMEM((2, 3, C), jnp.float32),
            pltpu.SemaphoreType.DMA((2,)),
            pltpu.SemaphoreType.DMA((2,)),
        ],
        compiler_params=_SC_PARAMS,
    )(cbase, fx, fy, fz, table)

    return jnp.stack(rgb, axis=1)


# speedup vs baseline: 1.0994x; 1.0994x over previous
import functools

import jax
import jax.numpy as jnp
from jax import lax
from jax.experimental import pallas as pl
from jax.experimental.pallas import tpu as pltpu
from jax.experimental.pallas import tpu_sc as plsc

R = 128
VMIN = -3.0
VMAX = 3.0

NC, NS, L = 2, 16, 16
NW = NC * NS
V = R * R * R
TW = 16
CS = 2048
C = 2048

_SC_PARAMS = pltpu.CompilerParams(needs_layout_passes=False,
                                  use_tc_tiling_on_sc=False)


def _wid():
    return lax.axis_index("s") * NC + lax.axis_index("c")


_SLAB = CS + 136


def _build_body(src_hbm, tab_hbm, src_v, dst_v, sin, sout):
    rpw = V // NW
    base = _wid() * rpw
    nchunk = rpw // CS
    lanes = lax.iota(jnp.int32, L)

    def slab_start(a, dx, c):
        x = a // (R * R)
        yb = (a - x * R * R) // R
        xs = jnp.minimum(x + dx, R - 1)
        return pl.multiple_of(xs * (R * R * 3) + c * (R * R) + yb * R, 128)

    def stage(ci, sl):
        a = base + ci * CS
        for dx in (0, 1):
            for c in range(3):
                pltpu.async_copy(
                    src_hbm.at[pl.ds(slab_start(a, dx, c), _SLAB)],
                    src_v.at[sl, dx * 3 + c], sin.at[sl])

    def wait_stage(ci, sl):
        a = base + ci * CS
        for dx in (0, 1):
            for c in range(3):
                pltpu.make_async_copy(
                    src_hbm.at[pl.ds(slab_start(a, dx, c), _SLAB)],
                    src_v.at[sl, dx * 3 + c], sin.at[sl]).wait()

    def compute(ci, sl):
        a = base + ci * CS

        slv = jnp.full((L,), sl, jnp.int32)

        def yline(yl, _):
            for zg in range(8):
                zb = zg * L
                rows = (yl * 8 + zg) * L + lanes
                for dx in (0, 1):
                    for c in range(3):
                        s = dx * 3 + c
                        colw = jnp.full((L,), dx * 6 + c, jnp.int32)
                        for dy in (0, 1):
                            off = (yl + dy) * R + zb
                            v0 = src_v[sl, s, pl.ds(off, L)]
                            v1 = src_v[sl, s, pl.ds(off + 1, L)]
                            pair = plsc.bitcast(
                                plsc.pack(v0, v1,
                                          format=plsc.PackFormat.INTERLEAVED),
                                jnp.float32)
                            plsc.store_scatter(dst_v,
                                               [slv, rows, colw + 3 * dy],
                                               pair)
            return 0

        lax.fori_loop(0, CS // L // 8, yline, 0)
        pltpu.async_copy(dst_v.at[sl], tab_hbm.at[pl.ds(a, CS)], sout.at[sl])

    def wait_out(ci, sl):
        a = base + ci * CS
        pltpu.make_async_copy(dst_v.at[sl], tab_hbm.at[pl.ds(a, CS)],
                              sout.at[sl]).wait()

    stage(0, 0)

    def pipe_step(ci, _):
        sl = lax.rem(ci, 2)
        nsl = lax.rem(ci + 1, 2)

        @pl.when(ci + 1 < nchunk)
        def _():
            stage(ci + 1, nsl)

        wait_stage(ci, sl)

        @pl.when(ci >= 2)
        def _():
            wait_out(ci - 2, sl)

        compute(ci, sl)
        return 0

    lax.fori_loop(0, nchunk, pipe_step, 0)
    wait_out(nchunk - 2, lax.rem(nchunk - 2, 2))
    wait_out(nchunk - 1, lax.rem(nchunk - 1, 2))


def _sample_body(idx_hbm, fx_hbm, fy_hbm, fz_hbm, tab_hbm,
                 r_hbm, g_hbm, b_hbm,
                 idx_v, f_v, rows_v, out_v, sems, fsems, osems, *, bpw):
    wbase = _wid() * bpw
    nchunk = bpw // C
    lanes = lax.iota(jnp.int32, L)

    def stage_fire(ci, sl):
        base = wbase + ci * C
        pltpu.sync_copy(idx_hbm.at[pl.ds(base, C)], idx_v.at[sl])
        pltpu.async_copy(tab_hbm.at[idx_v.at[sl]], rows_v.at[sl],
                         sems.at[sl])
        for ch, hbm in enumerate((fx_hbm, fy_hbm, fz_hbm)):
            pltpu.async_copy(hbm.at[pl.ds(base, C)], f_v.at[sl, ch],
                             fsems.at[sl])

    def wait_gathers(ci, sl):
        base = wbase + ci * C
        pltpu.make_async_copy(tab_hbm.at[idx_v.at[sl]],
                              rows_v.at[sl], sems.at[sl]).wait()
        for ch, hbm in enumerate((fx_hbm, fy_hbm, fz_hbm)):
            pltpu.make_async_copy(hbm.at[pl.ds(base, C)], f_v.at[sl, ch],
                                  fsems.at[sl]).wait()

    def blend_write(ci, sl):
        base = wbase + ci * C
        slv = jnp.full((L,), sl, jnp.int32)

        def blend_group(g, _):
            o = g * L
            rows = o + lanes
            fx = f_v[sl, 0, pl.ds(o, L)]
            fy = f_v[sl, 1, pl.ds(o, L)]
            fz = f_v[sl, 2, pl.ds(o, L)]
            gz = 1 - fz
            gy = 1 - fy
            gx = 1 - fx
            for c in range(3):

                def zlerp(dx, dy):
                    word = plsc.load_gather(
                        rows_v, [slv, rows,
                                 jnp.full((L,), dx * 6 + dy * 3 + c,
                                          jnp.int32)])
                    z0, z1 = plsc.unpack(
                        plsc.bitcast(word, jnp.bfloat16),
                        format=plsc.PackFormat.INTERLEAVED)
                    return z0 * gz + z1 * fz

                c0 = zlerp(0, 0) * gy + zlerp(0, 1) * fy
                c1 = zlerp(1, 0) * gy + zlerp(1, 1) * fy
                out_v[sl, c, pl.ds(o, L)] = c0 * gx + c1 * fx
            return 0

        lax.fori_loop(0, C // L, blend_group, 0)
        for ch, hbm in enumerate((r_hbm, g_hbm, b_hbm)):
            pltpu.async_copy(out_v.at[sl, ch], hbm.at[pl.ds(base, C)],
                             osems.at[sl])

    def wait_out(ci, sl):
        base = wbase + ci * C
        for ch, hbm in enumerate((r_hbm, g_hbm, b_hbm)):
            pltpu.make_async_copy(out_v.at[sl, ch], hbm.at[pl.ds(base, C)],
                                  osems.at[sl]).wait()

    stage_fire(0, 0)

    def pipe_step(ci, _):
        sl = lax.rem(ci, 2)
        nsl = lax.rem(ci + 1, 2)

        @pl.when(ci + 1 < nchunk)
        def _():
            stage_fire(ci + 1, nsl)

        wait_gathers(ci, sl)

        @pl.when(ci >= 2)
        def _():
            wait_out(ci - 2, sl)

        blend_write(ci, sl)
        return 0

    lax.fori_loop(0, nchunk, pipe_step, 0)
    wait_out(nchunk - 2, lax.rem(nchunk - 2, 2))
    wait_out(nchunk - 1, lax.rem(nchunk - 1, 2))


def kernel(position, grid):
    b = position.shape[0]
    assert b % (NW * C) == 0
    bpw = b // NW
    mesh = plsc.VectorSubcoreMesh(core_axis_name="c", subcore_axis_name="s",
                                  num_cores=NC, num_subcores=NS)

    t = (position - VMIN) / (VMAX - VMIN) * (R - 1)
    t = jnp.clip(t, 0.0, R - 1 - 1e-6)
    i0 = jnp.floor(t).astype(jnp.int32)
    f = t - i0.astype(jnp.float32)
    cbase = (i0[:, 0] * R + i0[:, 1]) * R + i0[:, 2]
    fx, fy, fz = f[:, 0], f[:, 1], f[:, 2]

    gwords = jnp.concatenate(
        [jnp.transpose(grid, (0, 3, 1, 2)).reshape(-1),
         jnp.zeros((512,), jnp.float32)])

    table = pl.kernel(
        _build_body,
        out_type=jax.ShapeDtypeStruct((V, TW), jnp.float32),
        mesh=mesh,
        scratch_types=[
            pltpu.VMEM((2, 6, _SLAB), jnp.float32),
            pltpu.VMEM((2, CS, TW), jnp.float32),
            pltpu.SemaphoreType.DMA((2,)),
            pltpu.SemaphoreType.DMA((2,)),
        ],
        compiler_params=_SC_PARAMS,
    )(gwords)

    rgb = pl.kernel(
        functools.partial(_sample_body, bpw=bpw),
        out_type=[jax.ShapeDtypeStruct((b,), jnp.float32)] * 3,
        mesh=mesh,
        scratch_types=[
            pltpu.VMEM((2, C), jnp.int32),
            pltpu.VMEM((2, 3, C), jnp.float32),
            pltpu.VMEM((2, C, TW), jnp.float32),
            pltpu.VMEM((2, 3, C), jnp.float32),
            pltpu.SemaphoreType.DMA((2,)),
            pltpu.SemaphoreType.DMA((2,)),
            pltpu.SemaphoreType.DMA((2,)),
        ],
        compiler_params=_SC_PARAMS,
    )(cbase, fx, fy, fz, table)

    return jnp.stack(rgb, axis=1)


# speedup vs baseline: 1.1479x; 1.0441x over previous
import functools

import jax
import jax.numpy as jnp
from jax import lax
from jax.experimental import pallas as pl
from jax.experimental.pallas import tpu as pltpu
from jax.experimental.pallas import tpu_sc as plsc

R = 128
VMIN = -3.0
VMAX = 3.0

NC, NS, L = 2, 16, 16
NW = NC * NS
V = R * R * R
TW = 16
CS = 2048
C = 2048

_SC_PARAMS = pltpu.CompilerParams(needs_layout_passes=False,
                                  use_tc_tiling_on_sc=False)


def _wid():
    return lax.axis_index("s") * NC + lax.axis_index("c")


_SLAB = CS + 136


def _build_body(src_hbm, tab_hbm, src_v, dst_v, sin, sout):
    rpw = V // NW
    base = _wid() * rpw
    nchunk = rpw // CS
    lanes = lax.iota(jnp.int32, L)

    def slab_start(a, dx, c):
        x = a // (R * R)
        yb = (a - x * R * R) // R
        xs = jnp.minimum(x + dx, R - 1)
        return pl.multiple_of(xs * (R * R * 3) + c * (R * R) + yb * R, 128)

    def stage(ci, sl):
        a = base + ci * CS
        for dx in (0, 1):
            for c in range(3):
                pltpu.async_copy(
                    src_hbm.at[pl.ds(slab_start(a, dx, c), _SLAB)],
                    src_v.at[sl, dx * 3 + c], sin.at[sl])

    def wait_stage(ci, sl):
        a = base + ci * CS
        for dx in (0, 1):
            for c in range(3):
                pltpu.make_async_copy(
                    src_hbm.at[pl.ds(slab_start(a, dx, c), _SLAB)],
                    src_v.at[sl, dx * 3 + c], sin.at[sl]).wait()

    def compute(ci, sl):
        a = base + ci * CS

        slv = jnp.full((L,), sl, jnp.int32)

        def yline(yl, _):
            for zg in range(8):
                zb = zg * L
                rows = (yl * 8 + zg) * L + lanes
                for dx in (0, 1):
                    for c in range(3):
                        s = dx * 3 + c
                        colw = jnp.full((L,), dx * 6 + c, jnp.int32)
                        for dy in (0, 1):
                            off = (yl + dy) * R + zb
                            v0 = src_v[sl, s, pl.ds(off, L)]
                            v1 = src_v[sl, s, pl.ds(off + 1, L)]
                            pair = plsc.bitcast(
                                plsc.pack(v0, v1,
                                          format=plsc.PackFormat.INTERLEAVED),
                                jnp.float32)
                            plsc.store_scatter(dst_v,
                                               [slv, rows, colw + 3 * dy],
                                               pair)
            return 0

        lax.fori_loop(0, CS // L // 8, yline, 0)
        pltpu.async_copy(dst_v.at[sl], tab_hbm.at[pl.ds(a, CS)], sout.at[sl])

    def wait_out(ci, sl):
        a = base + ci * CS
        pltpu.make_async_copy(dst_v.at[sl], tab_hbm.at[pl.ds(a, CS)],
                              sout.at[sl]).wait()

    stage(0, 0)

    def pipe_step(ci, _):
        sl = lax.rem(ci, 2)
        nsl = lax.rem(ci + 1, 2)

        @pl.when(ci + 1 < nchunk)
        def _():
            stage(ci + 1, nsl)

        wait_stage(ci, sl)

        @pl.when(ci >= 2)
        def _():
            wait_out(ci - 2, sl)

        compute(ci, sl)
        return 0

    lax.fori_loop(0, nchunk, pipe_step, 0)
    wait_out(nchunk - 2, lax.rem(nchunk - 2, 2))
    wait_out(nchunk - 1, lax.rem(nchunk - 1, 2))


def _sample_body(idx_hbm, fx_hbm, fy_hbm, fz_hbm, tab_hbm,
                 r_hbm, g_hbm, b_hbm,
                 idx_v, f_v, rows_v, out_v, sems, fsems, osems, isems, *, bpw):
    wbase = _wid() * bpw
    nchunk = bpw // C
    lanes = lax.iota(jnp.int32, L)

    def prefetch_idx(ci, sl):
        base = wbase + ci * C
        pltpu.async_copy(idx_hbm.at[pl.ds(base, C)], idx_v.at[sl],
                         isems.at[sl])

    def fire_gather(ci, sl):
        base = wbase + ci * C
        pltpu.make_async_copy(idx_hbm.at[pl.ds(base, C)], idx_v.at[sl],
                              isems.at[sl]).wait()
        pltpu.async_copy(tab_hbm.at[idx_v.at[sl]], rows_v.at[sl],
                         sems.at[sl])

    def stage_f(ci, sl):
        base = wbase + ci * C
        for ch, hbm in enumerate((fx_hbm, fy_hbm, fz_hbm)):
            pltpu.async_copy(hbm.at[pl.ds(base, C)], f_v.at[sl, ch],
                             fsems.at[sl])

    def wait_gathers(ci, sl):
        base = wbase + ci * C
        pltpu.make_async_copy(tab_hbm.at[idx_v.at[sl]],
                              rows_v.at[sl], sems.at[sl]).wait()
        for ch, hbm in enumerate((fx_hbm, fy_hbm, fz_hbm)):
            pltpu.make_async_copy(hbm.at[pl.ds(base, C)], f_v.at[sl, ch],
                                  fsems.at[sl]).wait()

    def blend_write(ci, sl):
        base = wbase + ci * C
        slv = jnp.full((L,), sl, jnp.int32)

        def blend_group(g, _):
            o = g * L
            rows = o + lanes
            fx = f_v[sl, 0, pl.ds(o, L)]
            fy = f_v[sl, 1, pl.ds(o, L)]
            fz = f_v[sl, 2, pl.ds(o, L)]
            gz = 1 - fz
            gy = 1 - fy
            gx = 1 - fx
            for c in range(3):

                def zlerp(dx, dy):
                    word = plsc.load_gather(
                        rows_v, [slv, rows,
                                 jnp.full((L,), dx * 6 + dy * 3 + c,
                                          jnp.int32)])
                    z0, z1 = plsc.unpack(
                        plsc.bitcast(word, jnp.bfloat16),
                        format=plsc.PackFormat.INTERLEAVED)
                    return z0 * gz + z1 * fz

                c0 = zlerp(0, 0) * gy + zlerp(0, 1) * fy
                c1 = zlerp(1, 0) * gy + zlerp(1, 1) * fy
                out_v[sl, c, pl.ds(o, L)] = c0 * gx + c1 * fx
            return 0

        lax.fori_loop(0, C // L, blend_group, 0)
        for ch, hbm in enumerate((r_hbm, g_hbm, b_hbm)):
            pltpu.async_copy(out_v.at[sl, ch], hbm.at[pl.ds(base, C)],
                             osems.at[sl])

    def wait_out(ci, sl):
        base = wbase + ci * C
        for ch, hbm in enumerate((r_hbm, g_hbm, b_hbm)):
            pltpu.make_async_copy(out_v.at[sl, ch], hbm.at[pl.ds(base, C)],
                                  osems.at[sl]).wait()

    prefetch_idx(0, 0)
    prefetch_idx(1, 1)
    fire_gather(0, 0)
    stage_f(0, 0)

    def pipe_step(ci, _):
        sl = lax.rem(ci, 2)
        nsl = lax.rem(ci + 1, 2)

        wait_gathers(ci, sl)

        @pl.when(ci + 1 < nchunk)
        def _():
            fire_gather(ci + 1, nsl)
            stage_f(ci + 1, nsl)

        @pl.when(ci + 2 < nchunk)
        def _():
            prefetch_idx(ci + 2, sl)

        @pl.when(ci >= 2)
        def _():
            wait_out(ci - 2, sl)

        blend_write(ci, sl)
        return 0

    lax.fori_loop(0, nchunk, pipe_step, 0)
    wait_out(nchunk - 2, lax.rem(nchunk - 2, 2))
    wait_out(nchunk - 1, lax.rem(nchunk - 1, 2))


def kernel(position, grid):
    b = position.shape[0]
    assert b % (NW * C) == 0
    bpw = b // NW
    mesh = plsc.VectorSubcoreMesh(core_axis_name="c", subcore_axis_name="s",
                                  num_cores=NC, num_subcores=NS)

    t = (position - VMIN) / (VMAX - VMIN) * (R - 1)
    t = jnp.clip(t, 0.0, R - 1 - 1e-6)
    i0 = jnp.floor(t).astype(jnp.int32)
    f = t - i0.astype(jnp.float32)
    cbase = (i0[:, 0] * R + i0[:, 1]) * R + i0[:, 2]
    fx, fy, fz = f[:, 0], f[:, 1], f[:, 2]

    gwords = jnp.concatenate(
        [jnp.transpose(grid, (0, 3, 1, 2)).reshape(-1),
         jnp.zeros((512,), jnp.float32)])

    table = pl.kernel(
        _build_body,
        out_type=jax.ShapeDtypeStruct((V, TW), jnp.float32),
        mesh=mesh,
        scratch_types=[
            pltpu.VMEM((2, 6, _SLAB), jnp.float32),
            pltpu.VMEM((2, CS, TW), jnp.float32),
            pltpu.SemaphoreType.DMA((2,)),
            pltpu.SemaphoreType.DMA((2,)),
        ],
        compiler_params=_SC_PARAMS,
    )(gwords)

    rgb = pl.kernel(
        functools.partial(_sample_body, bpw=bpw),
        out_type=[jax.ShapeDtypeStruct((b,), jnp.float32)] * 3,
        mesh=mesh,
        scratch_types=[
            pltpu.VMEM((2, C), jnp.int32),
            pltpu.VMEM((2, 3, C), jnp.float32),
            pltpu.VMEM((2, C, TW), jnp.float32),
            pltpu.VMEM((2, 3, C), jnp.float32),
            pltpu.SemaphoreType.DMA((2,)),
            pltpu.SemaphoreType.DMA((2,)),
            pltpu.SemaphoreType.DMA((2,)),
            pltpu.SemaphoreType.DMA((2,)),
        ],
        compiler_params=_SC_PARAMS,
    )(cbase, fx, fy, fz, table)

    return jnp.stack(rgb, axis=1)
